# Initial kernel scaffold; baseline (speedup 1.0000x reference)
import jax
import jax.numpy as jnp
from jax.experimental import pallas as pl


def kernel(x, edge_index, W1, Wl1, b1, W2, Wl2, b2):
    raise NotImplementedError("write your pallas kernel here")


# speedup vs baseline: 5.5721x; 5.5721x over previous
import functools

import jax
import jax.numpy as jnp
from jax import lax
from jax.experimental import pallas as pl
from jax.experimental.pallas import tpu as pltpu
from jax.experimental.pallas import tpu_sc as plsc

N = 10000
E = 160000
D = 256
H = D // 2
NTILES = 16
EDGES_PER_TILE = E // NTILES
CHUNK = 125
NCHUNKS = EDGES_PER_TILE // CHUNK
ROWS_PER_TILE = 632
NPAD = ROWS_PER_TILE * NTILES


def _sc_agg_body(xlo, xhi, src_h, dst_h, zeros_h, lo_out, hi_out,
                 src_v, dst_v, rows_v, accum, sem):
    c = lax.axis_index("c")
    s = lax.axis_index("s")

    pltpu.sync_copy(src_h.at[s], src_v)
    pltpu.sync_copy(dst_h.at[s], dst_v)
    pltpu.sync_copy(zeros_h, accum.at[pl.ds(s * ROWS_PER_TILE, ROWS_PER_TILE)])
    plsc.subcore_barrier()

    def chunk_step(j, carry):
        @pl.when(c == 0)
        def _():
            pltpu.async_copy(xlo.at[src_v.at[j]], rows_v, sem).wait()

        @pl.when(c == 1)
        def _():
            pltpu.async_copy(xhi.at[src_v.at[j]], rows_v, sem).wait()

        pltpu.sync_copy(rows_v, accum.at[dst_v.at[j]], add=True)
        return carry

    lax.fori_loop(0, NCHUNKS, chunk_step, 0)
    plsc.subcore_barrier()

    row0 = s * ROWS_PER_TILE

    @pl.when(c == 0)
    def _():
        pltpu.sync_copy(accum.at[pl.ds(row0, ROWS_PER_TILE)],
                        lo_out.at[pl.ds(row0, ROWS_PER_TILE)])

    @pl.when(c == 1)
    def _():
        pltpu.sync_copy(accum.at[pl.ds(row0, ROWS_PER_TILE)],
                        hi_out.at[pl.ds(row0, ROWS_PER_TILE)])


_sc_agg = functools.partial(
    pl.kernel,
    mesh=plsc.VectorSubcoreMesh(core_axis_name="c", subcore_axis_name="s"),
    out_type=(jax.ShapeDtypeStruct((NPAD, H), jnp.float32),
              jax.ShapeDtypeStruct((NPAD, H), jnp.float32)),
    scratch_types=[
        pltpu.VMEM((NCHUNKS, CHUNK), jnp.int32),
        pltpu.VMEM((NCHUNKS, CHUNK), jnp.int32),
        pltpu.VMEM((CHUNK, H), jnp.float32),
        pltpu.VMEM_SHARED((NPAD, H), jnp.float32),
        pltpu.SemaphoreType.DMA,
    ],
)(_sc_agg_body)


ROWS_BLK = 1000


def _mm1_body(alo_r, ahi_r, x_r, w1_r, wl1_r, b1_r, lo_r, hi_r):
    h = jnp.dot(alo_r[...], w1_r[:H, :], preferred_element_type=jnp.float32)
    h = h + jnp.dot(ahi_r[...], w1_r[H:, :], preferred_element_type=jnp.float32)
    h = h + jnp.dot(x_r[...], wl1_r[...], preferred_element_type=jnp.float32)
    h = h + b1_r[...]
    lo_r[...] = h[:, :H]
    hi_r[...] = h[:, H:]


def _mm2_body(alo_r, ahi_r, hlo_r, hhi_r, x_r, w2_r, wl2_r, b2_r, out_r):
    h = jnp.dot(alo_r[...], w2_r[:H, :], preferred_element_type=jnp.float32)
    h = h + jnp.dot(ahi_r[...], w2_r[H:, :], preferred_element_type=jnp.float32)
    h = h + jnp.dot(hlo_r[...], wl2_r[:H, :], preferred_element_type=jnp.float32)
    h = h + jnp.dot(hhi_r[...], wl2_r[H:, :], preferred_element_type=jnp.float32)
    h = h + b2_r[...]
    out_r[...] = (x_r[...] + h) * 0.5


def _row_blk(i):
    return (i, 0)


def _full(i):
    return (0, 0)


_half_spec = pl.BlockSpec((ROWS_BLK, H), _row_blk)
_fullrow_spec = pl.BlockSpec((ROWS_BLK, D), _row_blk)
_w_spec = pl.BlockSpec((D, D), _full)
_b_spec = pl.BlockSpec((1, D), _full)

_mm1 = pl.pallas_call(
    _mm1_body,
    grid=(N // ROWS_BLK,),
    in_specs=[_half_spec, _half_spec, _fullrow_spec, _w_spec, _w_spec, _b_spec],
    out_specs=[_half_spec, _half_spec],
    out_shape=(jax.ShapeDtypeStruct((N, H), jnp.float32),
               jax.ShapeDtypeStruct((N, H), jnp.float32)),
)

_mm2 = pl.pallas_call(
    _mm2_body,
    grid=(N // ROWS_BLK,),
    in_specs=[_half_spec, _half_spec, _half_spec, _half_spec, _fullrow_spec,
              _w_spec, _w_spec, _b_spec],
    out_specs=_fullrow_spec,
    out_shape=jax.ShapeDtypeStruct((N, D), jnp.float32),
)


def kernel(x, edge_index, W1, Wl1, b1, W2, Wl2, b2):
    x_lo = x[:, :H]
    x_hi = x[:, H:]
    src_h = edge_index[0].reshape(NTILES, NCHUNKS, CHUNK)
    dst_h = edge_index[1].reshape(NTILES, NCHUNKS, CHUNK)
    zeros = jnp.zeros((ROWS_PER_TILE, H), jnp.float32)
    b1r = b1.reshape(1, D)
    b2r = b2.reshape(1, D)

    a1lo, a1hi = _sc_agg(x_lo, x_hi, src_h, dst_h, zeros)
    h1lo, h1hi = _mm1(a1lo[:N], a1hi[:N], x, W1, Wl1, b1r)
    a2lo, a2hi = _sc_agg(h1lo, h1hi, src_h, dst_h, zeros)
    return _mm2(a2lo[:N], a2hi[:N], h1lo, h1hi, x, W2, Wl2, b2r)
